# baseline (device time: 150583 ns/iter reference)
import jax
import jax.numpy as jnp
from jax import lax
from jax.experimental import pallas as pl
from jax.experimental.pallas import tpu as pltpu

N_DEV = 4
HQ = 32
HG = 8
DH = 128
SQ = 1024
SKV = 1024
DM = 1024
NP = 4
BLK = 64
PER = SQ // NP
SCALE = 0.08838834764831843
BF = jnp.bfloat16

_PK = [(p, k) for p in range(NP) for k in range(NP)]


def kernel(x, Wq, K_ext, V_ext, Wo):
    x16 = x.astype(BF)
    wq16 = Wq.astype(BF)
    wo16 = Wo.astype(BF)
    my_b = lax.axis_index("i")
    perm = jnp.asarray(
        [BLK * (p + 4 * k) + r for (p, k) in _PK for r in range(BLK)],
        dtype=jnp.int32)
    Kt = jnp.transpose(
        lax.dynamic_slice(K_ext, (my_b, 0, 0, 0), (1, SKV, HQ, DH))[0],
        (1, 0, 2))[:, perm, :].astype(BF)
    Vt = jnp.transpose(
        lax.dynamic_slice(V_ext, (my_b, 0, 0, 0), (1, SKV, HQ, DH))[0],
        (1, 0, 2))[:, perm, :].astype(BF)

    def body(x_ref, wq_ref, k_ref, v_ref, wo_ref, out_ref,
             comm_ref, kbuf, vbuf, ctx_ref, acc_ref,
             send_sems, recv_a, recv_b, w_sems, kv_sems):
        my_pos = lax.axis_index("i")
        left = lax.rem(my_pos + N_DEV - 1, N_DEV)
        right = lax.rem(my_pos + 1, N_DEV)

        wq_dma = pltpu.make_async_copy(wq_ref, comm_ref.at[0, 0], w_sems.at[0])
        wo_dma = pltpu.make_async_copy(wo_ref, comm_ref.at[0, 1], w_sems.at[1])
        wq_dma.start()
        wo_dma.start()

        gs = [my_pos, left, right, lax.rem(my_pos + 2, N_DEV)]

        def kv_start(s):
            g = gs[s]
            pltpu.make_async_copy(
                k_ref.at[pl.ds(g * HG, HG)], kbuf.at[s % 2],
                kv_sems.at[0, s % 2]).start()
            pltpu.make_async_copy(
                v_ref.at[pl.ds(g * HG, HG)], vbuf.at[s % 2],
                kv_sems.at[1, s % 2]).start()

        def kv_wait(s):
            pltpu.make_async_copy(
                kbuf.at[s % 2], kbuf.at[s % 2], kv_sems.at[0, s % 2]).wait()
            pltpu.make_async_copy(
                vbuf.at[s % 2], vbuf.at[s % 2], kv_sems.at[1, s % 2]).wait()

        kv_start(0)

        barrier_sem = pltpu.get_barrier_semaphore()
        for nbr in (left, right):
            pl.semaphore_signal(
                barrier_sem, inc=1,
                device_id=(nbr,), device_id_type=pl.DeviceIdType.MESH,
            )
        pl.semaphore_wait(barrier_sem, 2)

        wq_dma.wait()
        wo_dma.wait()

        a_right = pltpu.make_async_remote_copy(
            src_ref=comm_ref.at[0], dst_ref=comm_ref.at[1],
            send_sem=send_sems.at[0], recv_sem=recv_a.at[0],
            device_id=(right,), device_id_type=pl.DeviceIdType.MESH,
        )
        a_left = pltpu.make_async_remote_copy(
            src_ref=comm_ref.at[0], dst_ref=comm_ref.at[2],
            send_sem=send_sems.at[1], recv_sem=recv_a.at[1],
            device_id=(left,), device_id_type=pl.DeviceIdType.MESH,
        )
        a_right.start()
        a_left.start()

        b_right = pltpu.make_async_remote_copy(
            src_ref=comm_ref.at[1, 0], dst_ref=comm_ref.at[3, 0],
            send_sem=send_sems.at[2], recv_sem=recv_b.at[0],
            device_id=(right,), device_id_type=pl.DeviceIdType.MESH,
        )
        b_left = pltpu.make_async_remote_copy(
            src_ref=comm_ref.at[2, 1], dst_ref=comm_ref.at[3, 1],
            send_sem=send_sems.at[3], recv_sem=recv_b.at[1],
            device_id=(left,), device_id_type=pl.DeviceIdType.MESH,
        )

        xp = jnp.concatenate(
            [x_ref[0, pl.ds(BLK * (p + 4 * k), BLK), :] for (p, k) in _PK],
            axis=0)

        def compute(s, slot):
            q = jnp.dot(xp, comm_ref[slot, 0],
                        preferred_element_type=jnp.float32).astype(BF)
            kv_wait(s)
            if s < 3:
                kv_start(s + 1)
            for hh in range(HG):
                qh = q[:, hh * DH:(hh + 1) * DH].reshape(NP, PER, DH)
                kh = kbuf[s % 2, hh].reshape(NP, PER, DH)
                sc = lax.dot_general(
                    qh, kh, (((2,), (2,)), ((0,), (0,))),
                    preferred_element_type=jnp.float32,
                ) * SCALE
                mx = jnp.max(sc, axis=2, keepdims=True)
                e = jnp.exp(sc - mx)
                w = (e / jnp.sum(e, axis=2, keepdims=True)).astype(BF)
                vh = vbuf[s % 2, hh].reshape(NP, PER, DH)
                ctxh = lax.dot_general(
                    w, vh, (((2,), (1,)), ((0,), (0,))),
                    preferred_element_type=jnp.float32)
                ctx_ref[:, hh * DH:(hh + 1) * DH] = (
                    ctxh.reshape(SQ, DH).astype(BF))
            part = jnp.dot(ctx_ref[:, :], comm_ref[slot, 1],
                           preferred_element_type=jnp.float32)
            if s == 0:
                acc_ref[:, :] = part
            elif s < 3:
                acc_ref[:, :] = acc_ref[:, :] + part
            else:
                full = acc_ref[:, :] + part
                for (p, k) in _PK:
                    out_ref[0, pl.ds(BLK * (p + 4 * k), BLK), :] = (
                        full[PER * p + BLK * k:PER * p + BLK * (k + 1), :])

        compute(0, 0)

        a_right.wait_recv()
        b_right.start()
        compute(1, 1)

        a_left.wait_recv()
        b_left.start()
        compute(2, 2)

        b_right.wait_recv()
        b_left.wait_recv()
        compute(3, 3)

        a_right.wait_send()
        a_left.wait_send()
        b_right.wait_send()
        b_left.wait_send()

    return pl.pallas_call(
        body,
        out_shape=jax.ShapeDtypeStruct((1, SQ, DM), jnp.float32),
        in_specs=[
            pl.BlockSpec(memory_space=pltpu.VMEM),
            pl.BlockSpec(memory_space=pl.ANY),
            pl.BlockSpec(memory_space=pl.ANY),
            pl.BlockSpec(memory_space=pl.ANY),
            pl.BlockSpec(memory_space=pl.ANY),
        ],
        out_specs=pl.BlockSpec(memory_space=pltpu.VMEM),
        scratch_shapes=[
            pltpu.VMEM((4, 2, DM, DM), BF),
            pltpu.VMEM((2, HG, SKV, DH), BF),
            pltpu.VMEM((2, HG, SKV, DH), BF),
            pltpu.VMEM((SQ, DM), BF),
            pltpu.VMEM((SQ, DM), jnp.float32),
            pltpu.SemaphoreType.DMA((4,)),
            pltpu.SemaphoreType.DMA((2,)),
            pltpu.SemaphoreType.DMA((2,)),
            pltpu.SemaphoreType.DMA((2,)),
            pltpu.SemaphoreType.DMA((2, 2)),
        ],
        compiler_params=pltpu.CompilerParams(
            collective_id=0, vmem_limit_bytes=60 * 1024 * 1024,
        ),
    )(x16, wq16, Kt, Vt, wo16)


# device time: 106748 ns/iter; 1.4106x vs baseline; 1.4106x over previous
import jax
import jax.numpy as jnp
from jax import lax
from jax.experimental import pallas as pl
from jax.experimental.pallas import tpu as pltpu

N_DEV = 4
HQ = 32
HG = 8
DH = 128
SQ = 1024
SKV = 1024
DM = 1024
NP = 4
BLK = 64
PER = SQ // NP
HALF = DM // 2
SCALE = 0.08838834764831843
BF = jnp.bfloat16

_PK = [(p, k) for p in range(NP) for k in range(NP)]


def kernel(x, Wq, K_ext, V_ext, Wo):
    x16 = x.astype(BF)
    wq16 = Wq.astype(BF)
    wo16 = Wo.astype(BF)

    def body(x_ref, wq_ref, k_ref, v_ref, wo_ref, out_ref,
             comm_ref, kbuf, vbuf, ctx_ref,
             send_sems, ra1, ra2, rb1, rb2, w_sems, kv_sems):
        my_pos = lax.axis_index("i")
        my_b = my_pos
        left = lax.rem(my_pos + N_DEV - 1, N_DEV)
        right = lax.rem(my_pos + 1, N_DEV)

        wq_dma = pltpu.make_async_copy(wq_ref, comm_ref.at[0, 0], w_sems.at[0])
        wo_dma = pltpu.make_async_copy(wo_ref, comm_ref.at[0, 1], w_sems.at[1])
        wq_dma.start()
        wo_dma.start()

        gs = [my_pos, left, right, lax.rem(my_pos + 2, N_DEV)]

        def kv_start(s):
            g = gs[s]
            for (p, k) in _PK:
                pltpu.make_async_copy(
                    k_ref.at[my_b, pl.ds(BLK * (p + 4 * k), BLK),
                             pl.ds(g * HG, HG), :],
                    kbuf.at[s % 2, pl.ds(PER * p + BLK * k, BLK)],
                    kv_sems.at[0, s % 2]).start()
                pltpu.make_async_copy(
                    v_ref.at[my_b, pl.ds(BLK * (p + 4 * k), BLK),
                             pl.ds(g * HG, HG), :],
                    vbuf.at[s % 2, pl.ds(PER * p + BLK * k, BLK)],
                    kv_sems.at[1, s % 2]).start()

        def kv_wait(s):
            pltpu.make_async_copy(
                kbuf.at[s % 2], kbuf.at[s % 2], kv_sems.at[0, s % 2]).wait()
            pltpu.make_async_copy(
                vbuf.at[s % 2], vbuf.at[s % 2], kv_sems.at[1, s % 2]).wait()

        kv_start(0)

        barrier_sem = pltpu.get_barrier_semaphore()
        for nbr in (left, right):
            pl.semaphore_signal(
                barrier_sem, inc=1,
                device_id=(nbr,), device_id_type=pl.DeviceIdType.MESH,
            )
        pl.semaphore_wait(barrier_sem, 2)

        wq_dma.wait()
        wo_dma.wait()

        def rdma(src, dst, s_ix, r_sem, dev):
            return pltpu.make_async_remote_copy(
                src_ref=src, dst_ref=dst,
                send_sem=send_sems.at[s_ix], recv_sem=r_sem,
                device_id=(dev,), device_id_type=pl.DeviceIdType.MESH)

        a1r = rdma(comm_ref.at[0, 0], comm_ref.at[1, 0], 0, ra1.at[0], right)
        a1l = rdma(comm_ref.at[0, 0], comm_ref.at[2, 0], 1, ra1.at[1], left)
        a2r = rdma(comm_ref.at[0, 1], comm_ref.at[1, 1], 2, ra2.at[0], right)
        a2l = rdma(comm_ref.at[0, 1], comm_ref.at[2, 1], 3, ra2.at[1], left)
        b1r = rdma(comm_ref.at[1, 0, :, pl.ds(0, HALF)],
                   comm_ref.at[3, 0, :, pl.ds(0, HALF)], 4, rb1.at[0], right)
        b1l = rdma(comm_ref.at[2, 0, :, pl.ds(HALF, HALF)],
                   comm_ref.at[3, 0, :, pl.ds(HALF, HALF)], 5, rb1.at[1], left)
        b2r = rdma(comm_ref.at[1, 1, pl.ds(0, HALF), :],
                   comm_ref.at[3, 1, pl.ds(0, HALF), :], 6, rb2.at[0], right)
        b2l = rdma(comm_ref.at[2, 1, pl.ds(HALF, HALF), :],
                   comm_ref.at[3, 1, pl.ds(HALF, HALF), :], 7, rb2.at[1], left)

        a1r.start()
        a1l.start()

        xp = jnp.concatenate(
            [x_ref[0, pl.ds(BLK * (p + 4 * k), BLK), :] for (p, k) in _PK],
            axis=0)

        def attn(s):
            q = (jnp.dot(xp, comm_ref[s, 0],
                         preferred_element_type=jnp.float32)
                 * SCALE).astype(BF)
            kv_wait(s)
            if s < 3:
                kv_start(s + 1)
            for hh in range(HG):
                qh = q[:, hh * DH:(hh + 1) * DH].reshape(NP, PER, DH)
                kh = kbuf[s % 2, :, hh, :].astype(BF).reshape(NP, PER, DH)
                sc = lax.dot_general(
                    qh, kh, (((2,), (2,)), ((0,), (0,))),
                    preferred_element_type=jnp.float32)
                e = jnp.exp(sc)
                w = (e / jnp.sum(e, axis=2, keepdims=True)).astype(BF)
                vh = vbuf[s % 2, :, hh, :].astype(BF).reshape(NP, PER, DH)
                ctxh = lax.dot_general(
                    w, vh, (((2,), (1,)), ((0,), (0,))),
                    preferred_element_type=jnp.float32)
                ctx_ref[s, :, hh * DH:(hh + 1) * DH] = (
                    ctxh.reshape(SQ, DH).astype(BF))

        def part(s):
            pt = jnp.dot(ctx_ref[s], comm_ref[s, 1],
                         preferred_element_type=jnp.float32)
            for (p, k) in _PK:
                blk = pt[PER * p + BLK * k:PER * p + BLK * (k + 1), :]
                dst = pl.ds(BLK * (p + 4 * k), BLK)
                if s == 0:
                    out_ref[0, dst, :] = blk
                else:
                    out_ref[0, dst, :] = out_ref[0, dst, :] + blk

        attn(0)
        part(0)

        a1r.wait_recv()
        b1r.start()
        a2r.start()
        a1l.wait_recv()
        b1l.start()
        a2l.start()

        attn(1)
        attn(2)

        b1r.wait_recv()
        b1l.wait_recv()
        attn(3)

        a2r.wait_recv()
        b2r.start()
        a2l.wait_recv()
        b2l.start()
        part(1)
        part(2)
        b2r.wait_recv()
        b2l.wait_recv()
        part(3)

        for d in (a1r, a1l, a2r, a2l, b1r, b1l, b2r, b2l):
            d.wait_send()

    return pl.pallas_call(
        body,
        out_shape=jax.ShapeDtypeStruct((1, SQ, DM), jnp.float32),
        in_specs=[
            pl.BlockSpec(memory_space=pltpu.VMEM),
            pl.BlockSpec(memory_space=pl.ANY),
            pl.BlockSpec(memory_space=pl.ANY),
            pl.BlockSpec(memory_space=pl.ANY),
            pl.BlockSpec(memory_space=pl.ANY),
        ],
        out_specs=pl.BlockSpec(memory_space=pltpu.VMEM),
        scratch_shapes=[
            pltpu.VMEM((4, 2, DM, DM), BF),
            pltpu.VMEM((2, SKV, HG, DH), jnp.float32),
            pltpu.VMEM((2, SKV, HG, DH), jnp.float32),
            pltpu.VMEM((N_DEV, SQ, DM), BF),
            pltpu.SemaphoreType.DMA((8,)),
            pltpu.SemaphoreType.DMA((2,)),
            pltpu.SemaphoreType.DMA((2,)),
            pltpu.SemaphoreType.DMA((2,)),
            pltpu.SemaphoreType.DMA((2,)),
            pltpu.SemaphoreType.DMA((2,)),
            pltpu.SemaphoreType.DMA((2, 2)),
        ],
        compiler_params=pltpu.CompilerParams(
            collective_id=0, vmem_limit_bytes=60 * 1024 * 1024,
        ),
    )(x16, wq16, K_ext, V_ext, wo16)


# device time: 103923 ns/iter; 1.4490x vs baseline; 1.0272x over previous
import jax
import jax.numpy as jnp
from jax import lax
from jax.experimental import pallas as pl
from jax.experimental.pallas import tpu as pltpu

N_DEV = 4
HQ = 32
HG = 8
DH = 128
SQ = 1024
SKV = 1024
DM = 1024
NP = 4
BLK = 64
PER = SQ // NP
HALF = DM // 2
SCALE = 0.08838834764831843
BF = jnp.bfloat16

_PK = [(p, k) for p in range(NP) for k in range(NP)]


def kernel(x, Wq, K_ext, V_ext, Wo):
    x16 = x.astype(BF)
    wq16 = Wq.astype(BF)
    wo16 = Wo.astype(BF)

    def body(x_ref, wq_ref, k_ref, v_ref, wo_ref, out_ref,
             comm_ref, kbuf, vbuf, khd, vhd, ctx_ref,
             send_sems, ra1, ra2, rb1, rb2, w_sems, kv_sems, hd_sems):
        my_pos = lax.axis_index("i")
        my_b = my_pos
        left = lax.rem(my_pos + N_DEV - 1, N_DEV)
        right = lax.rem(my_pos + 1, N_DEV)

        wq_dma = pltpu.make_async_copy(wq_ref, comm_ref.at[0, 0], w_sems.at[0])
        wo_dma = pltpu.make_async_copy(wo_ref, comm_ref.at[0, 1], w_sems.at[1])
        wq_dma.start()
        wo_dma.start()

        gs = [my_pos, left, right, lax.rem(my_pos + 2, N_DEV)]

        def kv_start(s):
            g = gs[s]
            for (p, k) in _PK:
                pltpu.make_async_copy(
                    k_ref.at[my_b, pl.ds(BLK * (p + 4 * k), BLK),
                             pl.ds(g * HG, HG), :],
                    kbuf.at[pl.ds(PER * p + BLK * k, BLK)],
                    kv_sems.at[0]).start()
                pltpu.make_async_copy(
                    v_ref.at[my_b, pl.ds(BLK * (p + 4 * k), BLK),
                             pl.ds(g * HG, HG), :],
                    vbuf.at[pl.ds(PER * p + BLK * k, BLK)],
                    kv_sems.at[1]).start()

        def kv_wait():
            pltpu.make_async_copy(kbuf, kbuf, kv_sems.at[0]).wait()
            pltpu.make_async_copy(vbuf, vbuf, kv_sems.at[1]).wait()

        def hd_repack():
            for hh in range(HG):
                pltpu.make_async_copy(
                    kbuf.at[:, hh, :], khd.at[hh], hd_sems.at[0]).start()
                pltpu.make_async_copy(
                    vbuf.at[:, hh, :], vhd.at[hh], hd_sems.at[1]).start()
            pltpu.make_async_copy(khd, khd, hd_sems.at[0]).wait()
            pltpu.make_async_copy(vhd, vhd, hd_sems.at[1]).wait()

        kv_start(0)

        barrier_sem = pltpu.get_barrier_semaphore()
        for nbr in (left, right):
            pl.semaphore_signal(
                barrier_sem, inc=1,
                device_id=(nbr,), device_id_type=pl.DeviceIdType.MESH,
            )
        pl.semaphore_wait(barrier_sem, 2)

        wq_dma.wait()
        wo_dma.wait()

        def rdma(src, dst, s_ix, r_sem, dev):
            return pltpu.make_async_remote_copy(
                src_ref=src, dst_ref=dst,
                send_sem=send_sems.at[s_ix], recv_sem=r_sem,
                device_id=(dev,), device_id_type=pl.DeviceIdType.MESH)

        a1r = rdma(comm_ref.at[0, 0], comm_ref.at[1, 0], 0, ra1.at[0], right)
        a1l = rdma(comm_ref.at[0, 0], comm_ref.at[2, 0], 1, ra1.at[1], left)
        a2r = rdma(comm_ref.at[0, 1], comm_ref.at[1, 1], 2, ra2.at[0], right)
        a2l = rdma(comm_ref.at[0, 1], comm_ref.at[2, 1], 3, ra2.at[1], left)
        b1r = rdma(comm_ref.at[1, 0, :, pl.ds(0, HALF)],
                   comm_ref.at[3, 0, :, pl.ds(0, HALF)], 4, rb1.at[0], right)
        b1l = rdma(comm_ref.at[2, 0, :, pl.ds(HALF, HALF)],
                   comm_ref.at[3, 0, :, pl.ds(HALF, HALF)], 5, rb1.at[1], left)
        b2r = rdma(comm_ref.at[1, 1, pl.ds(0, HALF), :],
                   comm_ref.at[3, 1, pl.ds(0, HALF), :], 6, rb2.at[0], right)
        b2l = rdma(comm_ref.at[2, 1, pl.ds(HALF, HALF), :],
                   comm_ref.at[3, 1, pl.ds(HALF, HALF), :], 7, rb2.at[1], left)

        a1r.start()
        a1l.start()

        xp = jnp.concatenate(
            [x_ref[0, pl.ds(BLK * (p + 4 * k), BLK), :] for (p, k) in _PK],
            axis=0)

        def attn(s):
            q = (jnp.dot(xp, comm_ref[s, 0],
                         preferred_element_type=jnp.float32)
                 * SCALE).astype(BF)
            kv_wait()
            hd_repack()
            if s < 3:
                kv_start(s + 1)
            for hh in range(HG):
                qh = q[:, hh * DH:(hh + 1) * DH].reshape(NP, PER, DH)
                kh = khd[hh].astype(BF).reshape(NP, PER, DH)
                sc = lax.dot_general(
                    qh, kh, (((2,), (2,)), ((0,), (0,))),
                    preferred_element_type=jnp.float32)
                e = jnp.exp(sc)
                w = (e / jnp.sum(e, axis=2, keepdims=True)).astype(BF)
                vh = vhd[hh].astype(BF).reshape(NP, PER, DH)
                ctxh = lax.dot_general(
                    w, vh, (((2,), (1,)), ((0,), (0,))),
                    preferred_element_type=jnp.float32)
                ctx_ref[s, :, hh * DH:(hh + 1) * DH] = (
                    ctxh.reshape(SQ, DH).astype(BF))

        def part(s):
            pt = jnp.dot(ctx_ref[s], comm_ref[s, 1],
                         preferred_element_type=jnp.float32)
            for (p, k) in _PK:
                blk = pt[PER * p + BLK * k:PER * p + BLK * (k + 1), :]
                dst = pl.ds(BLK * (p + 4 * k), BLK)
                if s == 0:
                    out_ref[0, dst, :] = blk
                else:
                    out_ref[0, dst, :] = out_ref[0, dst, :] + blk

        attn(0)
        part(0)

        a1r.wait_recv()
        b1r.start()
        a2r.start()
        a1l.wait_recv()
        b1l.start()
        a2l.start()

        attn(1)
        attn(2)

        b1r.wait_recv()
        b1l.wait_recv()
        attn(3)

        a2r.wait_recv()
        b2r.start()
        a2l.wait_recv()
        b2l.start()
        part(1)
        part(2)
        b2r.wait_recv()
        b2l.wait_recv()
        part(3)

        for d in (a1r, a1l, a2r, a2l, b1r, b1l, b2r, b2l):
            d.wait_send()

    return pl.pallas_call(
        body,
        out_shape=jax.ShapeDtypeStruct((1, SQ, DM), jnp.float32),
        in_specs=[
            pl.BlockSpec(memory_space=pltpu.VMEM),
            pl.BlockSpec(memory_space=pl.ANY),
            pl.BlockSpec(memory_space=pl.ANY),
            pl.BlockSpec(memory_space=pl.ANY),
            pl.BlockSpec(memory_space=pl.ANY),
        ],
        out_specs=pl.BlockSpec(memory_space=pltpu.VMEM),
        scratch_shapes=[
            pltpu.VMEM((4, 2, DM, DM), BF),
            pltpu.VMEM((SKV, HG, DH), jnp.float32),
            pltpu.VMEM((SKV, HG, DH), jnp.float32),
            pltpu.VMEM((HG, SKV, DH), jnp.float32),
            pltpu.VMEM((HG, SKV, DH), jnp.float32),
            pltpu.VMEM((N_DEV, SQ, DM), BF),
            pltpu.SemaphoreType.DMA((8,)),
            pltpu.SemaphoreType.DMA((2,)),
            pltpu.SemaphoreType.DMA((2,)),
            pltpu.SemaphoreType.DMA((2,)),
            pltpu.SemaphoreType.DMA((2,)),
            pltpu.SemaphoreType.DMA((2,)),
            pltpu.SemaphoreType.DMA((2,)),
            pltpu.SemaphoreType.DMA((2,)),
        ],
        compiler_params=pltpu.CompilerParams(
            collective_id=0, vmem_limit_bytes=60 * 1024 * 1024,
        ),
    )(x16, wq16, K_ext, V_ext, wo16)
